# Initial kernel scaffold; baseline (speedup 1.0000x reference)
import jax
import jax.numpy as jnp
from jax.experimental import pallas as pl


def kernel(feat, node_id, edge_index, params):
    raise NotImplementedError("write your pallas kernel here")


# speedup vs baseline: 1.0032x; 1.0032x over previous
import jax
import jax.numpy as jnp
from jax.experimental import pallas as pl
from jax.experimental.pallas import tpu as pltpu

T = 48
NNODES = 1140
N = T * NNODES
E = N * 16
D_FEAT = 43
EMB = 10
IN1 = D_FEAT + EMB
OUT1 = 48
OUT2 = 24
H = 64
LIN = 4 * H + IN1


def _bn(h, g, b):
    mu = h.mean(axis=0)
    var = h.var(axis=0)
    return (h - mu) / jnp.sqrt(var + 1e-5) * g + b


def _run_dir(xs, p, reverse):
    if reverse:
        xs = xs[::-1]
    B = xs.shape[1]

    def step(carry, xt):
        h, c = carry
        gates = xt @ p["Wih"].T + h @ p["Whh"].T + p["bih"] + p["bhh"]
        i, f, g, o = jnp.split(gates, 4, axis=1)
        i = jax.nn.sigmoid(i); f = jax.nn.sigmoid(f); g = jnp.tanh(g); o = jax.nn.sigmoid(o)
        c = f * c + i * g
        h = o * jnp.tanh(c)
        return (h, c), h

    init = (jnp.zeros((B, H), jnp.float32), jnp.zeros((B, H), jnp.float32))
    (hT, cT), hs = jax.lax.scan(step, init, xs)
    if reverse:
        hs = hs[::-1]
    return hs, hT


def _head_body(xcat_ref, g_ref, b_ref,
               aW1_ref, ab1_ref, aW2_ref, ab2_ref, aW3_ref, ab3_ref,
               cW1_ref, cb1_ref, cW2_ref, cb2_ref, cW3_ref, cb3_ref,
               act_ref, con_ref):
    x = xcat_ref[...]
    mu = jnp.mean(x, axis=0, keepdims=True)
    var = jnp.mean((x - mu) * (x - mu), axis=0, keepdims=True)
    x3 = jnp.tanh((x - mu) / jnp.sqrt(var + 1e-5) * g_ref[...] + b_ref[...])

    def head(w1, c1, w2, c2, w3, c3):
        h = jnp.tanh(jnp.dot(x3, w1[...], preferred_element_type=jnp.float32) + c1[...])
        h = jnp.tanh(jnp.dot(h, w2[...], preferred_element_type=jnp.float32) + c2[...])
        return jnp.dot(h, w3[...], preferred_element_type=jnp.float32) + c3[...]

    act_ref[...] = head(aW1_ref, ab1_ref, aW2_ref, ab2_ref, aW3_ref, ab3_ref)
    con_ref[...] = head(cW1_ref, cb1_ref, cW2_ref, cb2_ref, cW3_ref, cb3_ref)


def _heads(xcat, params):
    p = params
    args = [xcat,
            p["bn3_g"].reshape(1, LIN), p["bn3_b"].reshape(1, LIN),
            p["aW1"], p["ab1"].reshape(1, LIN), p["aW2"], p["ab2"].reshape(1, LIN),
            p["aW3"], p["ab3"].reshape(1, 1),
            p["cW1"], p["cb1"].reshape(1, LIN), p["cW2"], p["cb2"].reshape(1, LIN),
            p["cW3"], p["cb3"].reshape(1, 1)]
    return pl.pallas_call(
        _head_body,
        out_shape=[jax.ShapeDtypeStruct((NNODES, 1), jnp.float32),
                   jax.ShapeDtypeStruct((NNODES, 1), jnp.float32)],
    )(*args)


def kernel(feat, node_id, edge_index, params):
    x = jnp.concatenate([feat, params["emb"][node_id]], axis=1)
    src = jnp.concatenate([edge_index[0], jnp.arange(N)])
    dst = jnp.concatenate([edge_index[1], jnp.arange(N)])
    ones = jnp.ones((src.shape[0],), jnp.float32)
    deg_out = jnp.maximum(jax.ops.segment_sum(ones, src, N), 1.0)
    deg_in = jnp.maximum(jax.ops.segment_sum(ones, dst, N), 1.0)
    n_out = deg_out ** -0.5
    n_in = deg_in ** -0.5

    def gconv(h, W, b):
        h = h @ W
        m = h[src] * n_out[src][:, None]
        agg = jax.ops.segment_sum(m, dst, N)
        return agg * n_in[:, None] + b

    x1 = jnp.tanh(_bn(gconv(x, params["W1"], params["b1"]), params["bn1_g"], params["bn1_b"]))
    x2 = jnp.tanh(_bn(gconv(x1, params["W2"], params["b2"]), params["bn2_g"], params["bn2_b"]))
    inp = x2.reshape(T, NNODES, OUT2)
    h_finals = []
    for layer in params["lstm"]:
        hs_f, hf = _run_dir(inp, layer[0], False)
        hs_b, hb = _run_dir(inp, layer[1], True)
        inp = jnp.concatenate([hs_f, hs_b], axis=2)
        h_finals += [hf, hb]
    h_n = jnp.stack(h_finals, axis=0).reshape(NNODES, 4 * H)
    xcat = jnp.concatenate([x[-NNODES:], h_n], axis=1)
    active, consume = _heads(xcat, params)
    return (active, consume)


# speedup vs baseline: 11.1703x; 11.1352x over previous
import jax
import jax.numpy as jnp
from jax import lax
from jax.experimental import pallas as pl
from jax.experimental.pallas import tpu as pltpu
from jax.experimental.pallas import tpu_sc as plsc

T = 48
NNODES = 1140
N = T * NNODES
E = N * 16
D_FEAT = 43
EMB = 10
IN1 = D_FEAT + EMB
OUT1 = 48
OUT2 = 24
H = 64
LIN = 4 * H + IN1

NR = N + 64
RPT = NR // 16
KCH = 24
EP = 884736
NG1 = 18
NG2 = 9
NODEP = 57344
NPB = 14
BLK = 6840
NP = 1152

_f32 = jnp.float32


def _sc_mesh():
    return plsc.VectorSubcoreMesh(core_axis_name="c", subcore_axis_name="s")


def _sc_deg_emb(didxd, ones8, zeros8, nidp, emb):
    def body(didxd_h, ones_h, zeros_h, nid_h, emb_h, degs_h, embg_h,
             di, ones_v, nid_v, erows, acc8, sem):
        c = lax.axis_index("c")
        s = lax.axis_index("s")
        r0 = s * RPT
        pltpu.sync_copy(zeros_h.at[pl.ds(r0, RPT)], acc8.at[pl.ds(r0, RPT)])
        pltpu.sync_copy(ones_h, ones_v)
        plsc.subcore_barrier()

        def grp(g, carry):
            pltpu.sync_copy(didxd_h.at[c, s, g], di)
            for j in range(KCH):
                pltpu.sync_copy(ones_v, acc8.at[di.at[j]], add=True)
            return carry

        lax.fori_loop(0, NG1, grp, 0)
        plsc.subcore_barrier()
        pltpu.sync_copy(acc8.at[pl.ds(r0, RPT)], degs_h.at[c, pl.ds(r0, RPT)])

        w = c * 16 + s
        pltpu.sync_copy(nid_h.at[w], nid_v)
        for j in range(NPB):
            pltpu.async_copy(emb_h.at[nid_v.at[j]], erows, sem).wait()
            pltpu.sync_copy(erows, embg_h.at[pl.ds(w * NPB * 128 + j * 128, 128)])

    f = pl.kernel(
        body,
        out_type=[jax.ShapeDtypeStruct((2, NR, 8), _f32),
                  jax.ShapeDtypeStruct((NODEP, EMB), _f32)],
        mesh=_sc_mesh(),
        compiler_params=pltpu.CompilerParams(use_tc_tiling_on_sc=False),
        scratch_types=[
            pltpu.VMEM((KCH, 128), jnp.int32),
            pltpu.VMEM((128, 8), _f32),
            pltpu.VMEM((NPB, 128), jnp.int32),
            pltpu.VMEM((128, EMB), _f32),
            pltpu.VMEM_SHARED((NR, 8), _f32),
            pltpu.SemaphoreType.DMA,
        ],
    )
    return f(didxd, ones8, zeros8, nidp, emb)


def _sc_agg(table, gidx, didx, zeros24, ngrp):
    def body(table_h, gidx_h, didx_h, zeros_h, out_h, gi, di, rows, acc, sem):
        c = lax.axis_index("c")
        s = lax.axis_index("s")
        r0 = s * RPT
        pltpu.sync_copy(zeros_h.at[pl.ds(r0, RPT)], acc.at[pl.ds(r0, RPT)])
        plsc.subcore_barrier()

        def grp(g, carry):
            pltpu.sync_copy(gidx_h.at[c, s, g], gi)
            pltpu.sync_copy(didx_h.at[c, s, g], di)
            for j in range(KCH):
                pltpu.async_copy(table_h.at[gi.at[j]], rows, sem).wait()
                pltpu.sync_copy(rows, acc.at[di.at[j]], add=True)
            return carry

        lax.fori_loop(0, ngrp, grp, 0)
        plsc.subcore_barrier()
        pltpu.sync_copy(acc.at[pl.ds(r0, RPT)], out_h.at[c, pl.ds(r0, RPT)])

    f = pl.kernel(
        body,
        out_type=jax.ShapeDtypeStruct((2, NR, 24), _f32),
        mesh=_sc_mesh(),
        compiler_params=pltpu.CompilerParams(use_tc_tiling_on_sc=False),
        scratch_types=[
            pltpu.VMEM((KCH, 128), jnp.int32),
            pltpu.VMEM((KCH, 128), jnp.int32),
            pltpu.VMEM((128, 24), _f32),
            pltpu.VMEM_SHARED((NR, 24), _f32),
            pltpu.SemaphoreType.DMA,
        ],
    )
    return f(table, gidx, didx, zeros24)


def _tc_hn1(feat, embg, dego, W1):
    def body(feat_r, embg_r, dego_r, w_r, out_r):
        w = w_r[...]
        h = (jnp.dot(feat_r[...], w[0:D_FEAT, :], preferred_element_type=_f32)
             + jnp.dot(embg_r[...], w[D_FEAT:IN1, :], preferred_element_type=_f32))
        n_out = lax.rsqrt(dego_r[...][:, 0:1] + 1.0)
        out_r[...] = h * n_out

    return pl.pallas_call(
        body,
        grid=(8,),
        in_specs=[
            pl.BlockSpec((BLK, D_FEAT), lambda i: (i, 0)),
            pl.BlockSpec((BLK, EMB), lambda i: (i, 0)),
            pl.BlockSpec((BLK, 8), lambda i: (i, 0)),
            pl.BlockSpec((IN1, OUT1), lambda i: (0, 0)),
        ],
        out_specs=pl.BlockSpec((BLK, OUT1), lambda i: (i, 0)),
        out_shape=jax.ShapeDtypeStruct((N, OUT1), _f32),
    )(feat, embg, dego, W1)


def _o1(e0, e1, hn1, degi, b1):
    o = jnp.concatenate([e0, e1], axis=1) + hn1
    return o * lax.rsqrt(degi[:, 0:1] + 1.0) + b1


def _tc_stats1(e0, e1, hn1, degi, b1):
    def body(e0_r, e1_r, hn1_r, degi_r, b1_r, sum_r, sq_r):
        o = _o1(e0_r[...], e1_r[...], hn1_r[...], degi_r[...], b1_r[...])
        sum_r[...] = jnp.sum(o, axis=0, keepdims=True)[None]
        sq_r[...] = jnp.sum(o * o, axis=0, keepdims=True)[None]

    return pl.pallas_call(
        body,
        grid=(8,),
        in_specs=[
            pl.BlockSpec((BLK, 24), lambda i: (i, 0)),
            pl.BlockSpec((BLK, 24), lambda i: (i, 0)),
            pl.BlockSpec((BLK, OUT1), lambda i: (i, 0)),
            pl.BlockSpec((BLK, 8), lambda i: (i, 0)),
            pl.BlockSpec((1, OUT1), lambda i: (0, 0)),
        ],
        out_specs=[pl.BlockSpec((1, 1, OUT1), lambda i: (i, 0, 0)),
                   pl.BlockSpec((1, 1, OUT1), lambda i: (i, 0, 0))],
        out_shape=[jax.ShapeDtypeStruct((8, 1, OUT1), _f32),
                   jax.ShapeDtypeStruct((8, 1, OUT1), _f32)],
    )(e0, e1, hn1, degi, b1)


def _tc_apply1(e0, e1, hn1, degi, dego, b1, g1, bb1, W2, sums, sqs):
    def body(e0_r, e1_r, hn1_r, degi_r, dego_r, b1_r, g1_r, bb1_r, w2_r,
             sums_r, sqs_r, out_r):
        o = _o1(e0_r[...], e1_r[...], hn1_r[...], degi_r[...], b1_r[...])
        mu = jnp.sum(sums_r[...], axis=0) * (1.0 / N)
        msq = jnp.sum(sqs_r[...], axis=0) * (1.0 / N)
        var = msq - mu * mu
        x1 = jnp.tanh((o - mu) * lax.rsqrt(var + 1e-5) * g1_r[...] + bb1_r[...])
        h2 = jnp.dot(x1, w2_r[...], preferred_element_type=_f32)
        out_r[...] = h2 * lax.rsqrt(dego_r[...][:, 0:1] + 1.0)

    return pl.pallas_call(
        body,
        grid=(8,),
        in_specs=[
            pl.BlockSpec((BLK, 24), lambda i: (i, 0)),
            pl.BlockSpec((BLK, 24), lambda i: (i, 0)),
            pl.BlockSpec((BLK, OUT1), lambda i: (i, 0)),
            pl.BlockSpec((BLK, 8), lambda i: (i, 0)),
            pl.BlockSpec((BLK, 8), lambda i: (i, 0)),
            pl.BlockSpec((1, OUT1), lambda i: (0, 0)),
            pl.BlockSpec((1, OUT1), lambda i: (0, 0)),
            pl.BlockSpec((1, OUT1), lambda i: (0, 0)),
            pl.BlockSpec((OUT1, OUT2), lambda i: (0, 0)),
            pl.BlockSpec((8, 1, OUT1), lambda i: (0, 0, 0)),
            pl.BlockSpec((8, 1, OUT1), lambda i: (0, 0, 0)),
        ],
        out_specs=pl.BlockSpec((BLK, OUT2), lambda i: (i, 0)),
        out_shape=jax.ShapeDtypeStruct((N, OUT2), _f32),
    )(e0, e1, hn1, degi, dego, b1, g1, bb1, W2, sums, sqs)


def _o2(e0, e1, hn2, degi, b2):
    o = e0 + e1 + hn2
    return o * lax.rsqrt(degi[:, 0:1] + 1.0) + b2


def _tc_stats2(e0, e1, hn2, degi, b2):
    def body(e0_r, e1_r, hn2_r, degi_r, b2_r, sum_r, sq_r):
        o = _o2(e0_r[...], e1_r[...], hn2_r[...], degi_r[...], b2_r[...])
        sum_r[...] = jnp.sum(o, axis=0, keepdims=True)[None]
        sq_r[...] = jnp.sum(o * o, axis=0, keepdims=True)[None]

    return pl.pallas_call(
        body,
        grid=(8,),
        in_specs=[
            pl.BlockSpec((BLK, OUT2), lambda i: (i, 0)),
            pl.BlockSpec((BLK, OUT2), lambda i: (i, 0)),
            pl.BlockSpec((BLK, OUT2), lambda i: (i, 0)),
            pl.BlockSpec((BLK, 8), lambda i: (i, 0)),
            pl.BlockSpec((1, OUT2), lambda i: (0, 0)),
        ],
        out_specs=[pl.BlockSpec((1, 1, OUT2), lambda i: (i, 0, 0)),
                   pl.BlockSpec((1, 1, OUT2), lambda i: (i, 0, 0))],
        out_shape=[jax.ShapeDtypeStruct((8, 1, OUT2), _f32),
                   jax.ShapeDtypeStruct((8, 1, OUT2), _f32)],
    )(e0, e1, hn2, degi, b2)


def _tc_apply2(e0, e1, hn2, degi, b2, g2, bb2, sums, sqs):
    def body(e0_r, e1_r, hn2_r, degi_r, b2_r, g2_r, bb2_r, sums_r, sqs_r, out_r):
        o = _o2(e0_r[...], e1_r[...], hn2_r[...], degi_r[...], b2_r[...])
        mu = jnp.sum(sums_r[...], axis=0) * (1.0 / N)
        msq = jnp.sum(sqs_r[...], axis=0) * (1.0 / N)
        var = msq - mu * mu
        out_r[...] = jnp.tanh((o - mu) * lax.rsqrt(var + 1e-5) * g2_r[...] + bb2_r[...])

    return pl.pallas_call(
        body,
        grid=(8,),
        in_specs=[
            pl.BlockSpec((BLK, OUT2), lambda i: (i, 0)),
            pl.BlockSpec((BLK, OUT2), lambda i: (i, 0)),
            pl.BlockSpec((BLK, OUT2), lambda i: (i, 0)),
            pl.BlockSpec((BLK, 8), lambda i: (i, 0)),
            pl.BlockSpec((1, OUT2), lambda i: (0, 0)),
            pl.BlockSpec((1, OUT2), lambda i: (0, 0)),
            pl.BlockSpec((1, OUT2), lambda i: (0, 0)),
            pl.BlockSpec((8, 1, OUT2), lambda i: (0, 0, 0)),
            pl.BlockSpec((8, 1, OUT2), lambda i: (0, 0, 0)),
        ],
        out_specs=pl.BlockSpec((BLK, OUT2), lambda i: (i, 0)),
        out_shape=jax.ShapeDtypeStruct((N, OUT2), _f32),
    )(e0, e1, hn2, degi, b2, g2, bb2, sums, sqs)


def _tc_lstm(x2p, ws):
    w1fi, w1fh, b1f, w1bi, w1bh, b1b, w2fi, w2fh, b2f, w2bi, w2bh, b2b = ws

    def body(x_r, w1fi_r, w1fh_r, b1f_r, w1bi_r, w1bh_r, b1b_r,
             w2fi_r, w2fh_r, b2f_r, w2bi_r, w2bh_r, b2b_r, out_r, hs_r):
        def cell(gg, cc):
            ii = jax.nn.sigmoid(gg[:, 0:H])
            ff = jax.nn.sigmoid(gg[:, H:2 * H])
            g_ = jnp.tanh(gg[:, 2 * H:3 * H])
            oo = jax.nn.sigmoid(gg[:, 3 * H:4 * H])
            cn = ff * cc + ii * g_
            return oo * jnp.tanh(cn), cn

        z = jnp.zeros((NP, H), _f32)

        def step1(t, c4):
            hf, cf, hb, cb = c4
            tb = 47 - t
            dn = (((0,), (0,)), ((), ()))
            gf = (lax.dot_general(x_r[t], w1fi_r[...], dn, preferred_element_type=_f32)
                  + jnp.dot(hf, w1fh_r[...], preferred_element_type=_f32) + b1f_r[...])
            gb = (lax.dot_general(x_r[tb], w1bi_r[...], dn, preferred_element_type=_f32)
                  + jnp.dot(hb, w1bh_r[...], preferred_element_type=_f32) + b1b_r[...])
            hf, cf = cell(gf, cf)
            hb, cb = cell(gb, cb)
            hs_r[t, :, 0:H] = hf
            hs_r[tb, :, H:2 * H] = hb
            return (hf, cf, hb, cb)

        hf1, _, hb1, _ = lax.fori_loop(0, T, step1, (z, z, z, z))
        out_r[0] = hf1
        out_r[1] = hb1

        def step2(t, c4):
            hf, cf, hb, cb = c4
            tb = 47 - t
            gf = (jnp.dot(hs_r[t], w2fi_r[...], preferred_element_type=_f32)
                  + jnp.dot(hf, w2fh_r[...], preferred_element_type=_f32) + b2f_r[...])
            gb = (jnp.dot(hs_r[tb], w2bi_r[...], preferred_element_type=_f32)
                  + jnp.dot(hb, w2bh_r[...], preferred_element_type=_f32) + b2b_r[...])
            hf, cf = cell(gf, cf)
            hb, cb = cell(gb, cb)
            return (hf, cf, hb, cb)

        hf2, _, hb2, _ = lax.fori_loop(0, T, step2, (z, z, z, z))
        out_r[2] = hf2
        out_r[3] = hb2

    return pl.pallas_call(
        body,
        out_shape=jax.ShapeDtypeStruct((4, NP, H), _f32),
        scratch_shapes=[pltpu.VMEM((T, NP, 2 * H), _f32)],
    )(x2p, w1fi, w1fh, b1f, w1bi, w1bh, b1b, w2fi, w2fh, b2f, w2bi, w2bh, b2b)


def _head_body(xcat_ref, g_ref, b_ref,
               aW1_ref, ab1_ref, aW2_ref, ab2_ref, aW3_ref, ab3_ref,
               cW1_ref, cb1_ref, cW2_ref, cb2_ref, cW3_ref, cb3_ref,
               act_ref, con_ref):
    x = xcat_ref[...]
    mu = jnp.mean(x, axis=0, keepdims=True)
    var = jnp.mean((x - mu) * (x - mu), axis=0, keepdims=True)
    x3 = jnp.tanh((x - mu) / jnp.sqrt(var + 1e-5) * g_ref[...] + b_ref[...])

    def head(w1, c1, w2, c2, w3, c3):
        h = jnp.tanh(jnp.dot(x3, w1[...], preferred_element_type=_f32) + c1[...])
        h = jnp.tanh(jnp.dot(h, w2[...], preferred_element_type=_f32) + c2[...])
        return jnp.dot(h, w3[...], preferred_element_type=_f32) + c3[...]

    act_ref[...] = head(aW1_ref, ab1_ref, aW2_ref, ab2_ref, aW3_ref, ab3_ref)
    con_ref[...] = head(cW1_ref, cb1_ref, cW2_ref, cb2_ref, cW3_ref, cb3_ref)


def _heads(xcat, p):
    args = [xcat,
            p["bn3_g"].reshape(1, LIN), p["bn3_b"].reshape(1, LIN),
            p["aW1"], p["ab1"].reshape(1, LIN), p["aW2"], p["ab2"].reshape(1, LIN),
            p["aW3"], p["ab3"].reshape(1, 1),
            p["cW1"], p["cb1"].reshape(1, LIN), p["cW2"], p["cb2"].reshape(1, LIN),
            p["cW3"], p["cb3"].reshape(1, 1)]
    return pl.pallas_call(
        _head_body,
        out_shape=[jax.ShapeDtypeStruct((NNODES, 1), _f32),
                   jax.ShapeDtypeStruct((NNODES, 1), _f32)],
    )(*args)


def kernel(feat, node_id, edge_index, params):
    p = params
    src = edge_index[0].astype(jnp.int32)
    dst = edge_index[1].astype(jnp.int32)

    padn = EP - E
    pidx = jnp.arange(padn, dtype=jnp.int32)
    sfg = jnp.concatenate([src, pidx % 1024])
    ddm = jnp.concatenate([dst, N + (pidx % 64)])
    sdm = jnp.concatenate([src, N + (pidx % 64)])

    gidx1 = (2 * sfg)[None, :] + jnp.array([[0], [1]], jnp.int32)
    gidx1 = gidx1.reshape(2, 16, NG1, KCH, 128)
    didx1 = jnp.broadcast_to(ddm, (2, EP)).reshape(2, 16, NG1, KCH, 128)
    gidx2 = sfg.reshape(2, 16, NG2, KCH, 128)
    didx2 = ddm.reshape(2, 16, NG2, KCH, 128)
    didxd = jnp.stack([sdm, ddm]).reshape(2, 16, NG1, KCH, 128)

    nidp = jnp.concatenate(
        [node_id.astype(jnp.int32),
         jnp.arange(NODEP - N, dtype=jnp.int32) % NNODES]).reshape(32, NPB, 128)

    zeros24 = jnp.zeros((NR, 24), _f32)
    zeros8 = jnp.zeros((NR, 8), _f32)
    ones8 = jnp.ones((128, 8), _f32)

    degs, embg_full = _sc_deg_emb(didxd, ones8, zeros8, nidp, p["emb"])
    embg = embg_full[:N]
    dego = degs[0, :N]
    degi = degs[1, :N]

    hn1 = _tc_hn1(feat, embg, dego, p["W1"])

    eagg1 = _sc_agg(hn1.reshape(2 * N, 24), gidx1, didx1, zeros24, NG1)
    e10 = eagg1[0, :N]
    e11 = eagg1[1, :N]

    b1 = p["b1"].reshape(1, OUT1)
    s1, q1 = _tc_stats1(e10, e11, hn1, degi, b1)
    hn2 = _tc_apply1(e10, e11, hn1, degi, dego, b1,
                     p["bn1_g"].reshape(1, OUT1), p["bn1_b"].reshape(1, OUT1),
                     p["W2"], s1, q1)

    eagg2 = _sc_agg(hn2, gidx2, didx2, zeros24, NG2)
    e20 = eagg2[0, :N]
    e21 = eagg2[1, :N]

    b2 = p["b2"].reshape(1, OUT2)
    s2, q2 = _tc_stats2(e20, e21, hn2, degi, b2)
    x2 = _tc_apply2(e20, e21, hn2, degi, b2,
                    p["bn2_g"].reshape(1, OUT2), p["bn2_b"].reshape(1, OUT2),
                    s2, q2)

    x2p = jnp.pad(x2.reshape(T, NNODES, OUT2).transpose(0, 2, 1),
                  ((0, 0), (0, 0), (0, NP - NNODES)))

    lw = p["lstm"]
    ws = []
    for li in range(2):
        for d in range(2):
            pd = lw[li][d]
            ws += [pd["Wih"].T, pd["Whh"].T,
                   (pd["bih"] + pd["bhh"]).reshape(1, 4 * H)]
    finals = _tc_lstm(x2p, ws)
    h_n = finals[:, :NNODES, :].reshape(NNODES, 4 * H)

    x_last = jnp.concatenate([feat[-NNODES:], embg[-NNODES:]], axis=1)
    xcat = jnp.concatenate([x_last, h_n], axis=1)
    active, consume = _heads(xcat, p)
    return (active, consume)


# speedup vs baseline: 14.0847x; 1.2609x over previous
import jax
import jax.numpy as jnp
from jax import lax
from jax.experimental import pallas as pl
from jax.experimental.pallas import tpu as pltpu
from jax.experimental.pallas import tpu_sc as plsc

T = 48
NNODES = 1140
N = T * NNODES
E = N * 16
D_FEAT = 43
EMB = 10
IN1 = D_FEAT + EMB
OUT1 = 48
OUT2 = 24
H = 64
LIN = 4 * H + IN1

NR = N + 64
RPT = NR // 16
KCH = 12
EP = 884736
NG1 = 36
NG2 = 18
NODEP = 57344
NPB = 14
BLK = 6840
NP = 1152

_f32 = jnp.float32


def _sc_mesh():
    return plsc.VectorSubcoreMesh(core_axis_name="c", subcore_axis_name="s")


def _sc_deg_emb(didxd, ones8, zeros8, nidp, emb):
    def body(didxd_h, ones_h, zeros_h, nid_h, emb_h, degs_h, embg_h,
             di, ones_v, nid_v, *rest):
        erows = rest[:NPB]
        acc8, sem, ssem = rest[NPB], rest[NPB + 1], rest[NPB + 2]
        c = lax.axis_index("c")
        s = lax.axis_index("s")
        r0 = s * RPT
        pltpu.sync_copy(zeros_h.at[pl.ds(r0, RPT)], acc8.at[pl.ds(r0, RPT)])
        pltpu.sync_copy(ones_h, ones_v)
        plsc.subcore_barrier()

        w = c * 16 + s
        pltpu.sync_copy(nid_h.at[w], nid_v)
        gd = [pltpu.async_copy(emb_h.at[nid_v.at[j]], erows[j], sem)
              for j in range(NPB)]

        def grp(g, carry):
            pltpu.sync_copy(didxd_h.at[c, s, g], di)
            sd = [pltpu.async_copy(ones_v, acc8.at[di.at[j]], ssem, add=True)
                  for j in range(KCH)]
            for d_ in sd:
                d_.wait()
            return carry

        lax.fori_loop(0, NG1, grp, 0)
        plsc.subcore_barrier()
        pltpu.sync_copy(acc8.at[pl.ds(r0, RPT)], degs_h.at[c, pl.ds(r0, RPT)])

        for d_ in gd:
            d_.wait()
        sd = [pltpu.async_copy(erows[j],
                               embg_h.at[pl.ds(w * NPB * 128 + j * 128, 128)], sem)
              for j in range(NPB)]
        for d_ in sd:
            d_.wait()

    f = pl.kernel(
        body,
        out_type=[jax.ShapeDtypeStruct((2, NR, 8), _f32),
                  jax.ShapeDtypeStruct((NODEP, 16), _f32)],
        mesh=_sc_mesh(),
        compiler_params=pltpu.CompilerParams(use_tc_tiling_on_sc=False),
        scratch_types=[
            pltpu.VMEM((KCH, 128), jnp.int32),
            pltpu.VMEM((128, 8), _f32),
            pltpu.VMEM((NPB, 128), jnp.int32),
            *[pltpu.VMEM((128, 16), _f32) for _ in range(NPB)],
            pltpu.VMEM_SHARED((NR, 8), _f32),
            pltpu.SemaphoreType.DMA,
            pltpu.SemaphoreType.DMA,
        ],
    )
    return f(didxd, ones8, zeros8, nidp, emb)


def _sc_agg(table, gidx, didx, zeros24, ngrp):
    def body(table_h, gidx_h, didx_h, zeros_h, out_h, gi, di, *rest):
        rows = rest[:KCH]
        acc, sem, ssem = rest[KCH], rest[KCH + 1], rest[KCH + 2]
        c = lax.axis_index("c")
        s = lax.axis_index("s")
        r0 = s * RPT
        pltpu.sync_copy(zeros_h.at[pl.ds(r0, RPT)], acc.at[pl.ds(r0, RPT)])
        plsc.subcore_barrier()

        def grp(g, carry):
            pltpu.sync_copy(gidx_h.at[c, s, g], gi)
            pltpu.sync_copy(didx_h.at[c, s, g], di)
            gd = [pltpu.async_copy(table_h.at[gi.at[j]], rows[j], sem)
                  for j in range(KCH)]
            for d_ in gd:
                d_.wait()
            sd = [pltpu.async_copy(rows[j], acc.at[di.at[j]], ssem, add=True)
                  for j in range(KCH)]
            for d_ in sd:
                d_.wait()
            return carry

        lax.fori_loop(0, ngrp, grp, 0)
        plsc.subcore_barrier()
        pltpu.sync_copy(acc.at[pl.ds(r0, RPT)], out_h.at[c, pl.ds(r0, RPT)])

    f = pl.kernel(
        body,
        out_type=jax.ShapeDtypeStruct((2, NR, 24), _f32),
        mesh=_sc_mesh(),
        compiler_params=pltpu.CompilerParams(use_tc_tiling_on_sc=False),
        scratch_types=[
            pltpu.VMEM((KCH, 128), jnp.int32),
            pltpu.VMEM((KCH, 128), jnp.int32),
            *[pltpu.VMEM((128, 24), _f32) for _ in range(KCH)],
            pltpu.VMEM_SHARED((NR, 24), _f32),
            pltpu.SemaphoreType.DMA,
            pltpu.SemaphoreType.DMA,
        ],
    )
    return f(table, gidx, didx, zeros24)


def _tc_hn1(feat, embg, dego, W1):
    def body(feat_r, embg_r, dego_r, w_r, out_r):
        w = w_r[...]
        h = (jnp.dot(feat_r[...], w[0:D_FEAT, :], preferred_element_type=_f32)
             + jnp.dot(embg_r[...], w[D_FEAT:IN1, :], preferred_element_type=_f32))
        n_out = lax.rsqrt(dego_r[...][:, 0:1] + 1.0)
        out_r[...] = h * n_out

    return pl.pallas_call(
        body,
        grid=(8,),
        in_specs=[
            pl.BlockSpec((BLK, D_FEAT), lambda i: (i, 0)),
            pl.BlockSpec((BLK, EMB), lambda i: (i, 0)),
            pl.BlockSpec((BLK, 8), lambda i: (i, 0)),
            pl.BlockSpec((IN1, OUT1), lambda i: (0, 0)),
        ],
        out_specs=pl.BlockSpec((BLK, OUT1), lambda i: (i, 0)),
        out_shape=jax.ShapeDtypeStruct((N, OUT1), _f32),
    )(feat, embg, dego, W1)


def _o1(e0, e1, hn1, degi, b1):
    o = jnp.concatenate([e0, e1], axis=1) + hn1
    return o * lax.rsqrt(degi[:, 0:1] + 1.0) + b1


def _tc_stats1(e0, e1, hn1, degi, b1):
    def body(e0_r, e1_r, hn1_r, degi_r, b1_r, sum_r, sq_r):
        o = _o1(e0_r[...], e1_r[...], hn1_r[...], degi_r[...], b1_r[...])
        sum_r[...] = jnp.sum(o, axis=0, keepdims=True)[None]
        sq_r[...] = jnp.sum(o * o, axis=0, keepdims=True)[None]

    return pl.pallas_call(
        body,
        grid=(8,),
        in_specs=[
            pl.BlockSpec((BLK, 24), lambda i: (i, 0)),
            pl.BlockSpec((BLK, 24), lambda i: (i, 0)),
            pl.BlockSpec((BLK, OUT1), lambda i: (i, 0)),
            pl.BlockSpec((BLK, 8), lambda i: (i, 0)),
            pl.BlockSpec((1, OUT1), lambda i: (0, 0)),
        ],
        out_specs=[pl.BlockSpec((1, 1, OUT1), lambda i: (i, 0, 0)),
                   pl.BlockSpec((1, 1, OUT1), lambda i: (i, 0, 0))],
        out_shape=[jax.ShapeDtypeStruct((8, 1, OUT1), _f32),
                   jax.ShapeDtypeStruct((8, 1, OUT1), _f32)],
    )(e0, e1, hn1, degi, b1)


def _tc_apply1(e0, e1, hn1, degi, dego, b1, g1, bb1, W2, sums, sqs):
    def body(e0_r, e1_r, hn1_r, degi_r, dego_r, b1_r, g1_r, bb1_r, w2_r,
             sums_r, sqs_r, out_r):
        o = _o1(e0_r[...], e1_r[...], hn1_r[...], degi_r[...], b1_r[...])
        mu = jnp.sum(sums_r[...], axis=0) * (1.0 / N)
        msq = jnp.sum(sqs_r[...], axis=0) * (1.0 / N)
        var = msq - mu * mu
        x1 = jnp.tanh((o - mu) * lax.rsqrt(var + 1e-5) * g1_r[...] + bb1_r[...])
        h2 = jnp.dot(x1, w2_r[...], preferred_element_type=_f32)
        out_r[...] = h2 * lax.rsqrt(dego_r[...][:, 0:1] + 1.0)

    return pl.pallas_call(
        body,
        grid=(8,),
        in_specs=[
            pl.BlockSpec((BLK, 24), lambda i: (i, 0)),
            pl.BlockSpec((BLK, 24), lambda i: (i, 0)),
            pl.BlockSpec((BLK, OUT1), lambda i: (i, 0)),
            pl.BlockSpec((BLK, 8), lambda i: (i, 0)),
            pl.BlockSpec((BLK, 8), lambda i: (i, 0)),
            pl.BlockSpec((1, OUT1), lambda i: (0, 0)),
            pl.BlockSpec((1, OUT1), lambda i: (0, 0)),
            pl.BlockSpec((1, OUT1), lambda i: (0, 0)),
            pl.BlockSpec((OUT1, OUT2), lambda i: (0, 0)),
            pl.BlockSpec((8, 1, OUT1), lambda i: (0, 0, 0)),
            pl.BlockSpec((8, 1, OUT1), lambda i: (0, 0, 0)),
        ],
        out_specs=pl.BlockSpec((BLK, OUT2), lambda i: (i, 0)),
        out_shape=jax.ShapeDtypeStruct((N, OUT2), _f32),
    )(e0, e1, hn1, degi, dego, b1, g1, bb1, W2, sums, sqs)


def _o2(e0, e1, hn2, degi, b2):
    o = e0 + e1 + hn2
    return o * lax.rsqrt(degi[:, 0:1] + 1.0) + b2


def _tc_stats2(e0, e1, hn2, degi, b2):
    def body(e0_r, e1_r, hn2_r, degi_r, b2_r, sum_r, sq_r):
        o = _o2(e0_r[...], e1_r[...], hn2_r[...], degi_r[...], b2_r[...])
        sum_r[...] = jnp.sum(o, axis=0, keepdims=True)[None]
        sq_r[...] = jnp.sum(o * o, axis=0, keepdims=True)[None]

    return pl.pallas_call(
        body,
        grid=(8,),
        in_specs=[
            pl.BlockSpec((BLK, OUT2), lambda i: (i, 0)),
            pl.BlockSpec((BLK, OUT2), lambda i: (i, 0)),
            pl.BlockSpec((BLK, OUT2), lambda i: (i, 0)),
            pl.BlockSpec((BLK, 8), lambda i: (i, 0)),
            pl.BlockSpec((1, OUT2), lambda i: (0, 0)),
        ],
        out_specs=[pl.BlockSpec((1, 1, OUT2), lambda i: (i, 0, 0)),
                   pl.BlockSpec((1, 1, OUT2), lambda i: (i, 0, 0))],
        out_shape=[jax.ShapeDtypeStruct((8, 1, OUT2), _f32),
                   jax.ShapeDtypeStruct((8, 1, OUT2), _f32)],
    )(e0, e1, hn2, degi, b2)


def _tc_apply2(e0, e1, hn2, degi, b2, g2, bb2, sums, sqs):
    def body(e0_r, e1_r, hn2_r, degi_r, b2_r, g2_r, bb2_r, sums_r, sqs_r, out_r):
        o = _o2(e0_r[...], e1_r[...], hn2_r[...], degi_r[...], b2_r[...])
        mu = jnp.sum(sums_r[...], axis=0) * (1.0 / N)
        msq = jnp.sum(sqs_r[...], axis=0) * (1.0 / N)
        var = msq - mu * mu
        out_r[...] = jnp.tanh((o - mu) * lax.rsqrt(var + 1e-5) * g2_r[...] + bb2_r[...])

    return pl.pallas_call(
        body,
        grid=(8,),
        in_specs=[
            pl.BlockSpec((BLK, OUT2), lambda i: (i, 0)),
            pl.BlockSpec((BLK, OUT2), lambda i: (i, 0)),
            pl.BlockSpec((BLK, OUT2), lambda i: (i, 0)),
            pl.BlockSpec((BLK, 8), lambda i: (i, 0)),
            pl.BlockSpec((1, OUT2), lambda i: (0, 0)),
            pl.BlockSpec((1, OUT2), lambda i: (0, 0)),
            pl.BlockSpec((1, OUT2), lambda i: (0, 0)),
            pl.BlockSpec((8, 1, OUT2), lambda i: (0, 0, 0)),
            pl.BlockSpec((8, 1, OUT2), lambda i: (0, 0, 0)),
        ],
        out_specs=pl.BlockSpec((BLK, OUT2), lambda i: (i, 0)),
        out_shape=jax.ShapeDtypeStruct((N, OUT2), _f32),
    )(e0, e1, hn2, degi, b2, g2, bb2, sums, sqs)


def _tc_lstm(x2p, ws):
    w1fi, w1fh, b1f, w1bi, w1bh, b1b, w2fi, w2fh, b2f, w2bi, w2bh, b2b = ws

    def body(x_r, w1fi_r, w1fh_r, b1f_r, w1bi_r, w1bh_r, b1b_r,
             w2fi_r, w2fh_r, b2f_r, w2bi_r, w2bh_r, b2b_r, out_r, hs_r):
        def cell(gg, cc):
            ii = jax.nn.sigmoid(gg[:, 0:H])
            ff = jax.nn.sigmoid(gg[:, H:2 * H])
            g_ = jnp.tanh(gg[:, 2 * H:3 * H])
            oo = jax.nn.sigmoid(gg[:, 3 * H:4 * H])
            cn = ff * cc + ii * g_
            return oo * jnp.tanh(cn), cn

        z = jnp.zeros((NP, H), _f32)

        def step1(t, c4):
            hf, cf, hb, cb = c4
            tb = 47 - t
            dn = (((0,), (0,)), ((), ()))
            gf = (lax.dot_general(x_r[t], w1fi_r[...], dn, preferred_element_type=_f32)
                  + jnp.dot(hf, w1fh_r[...], preferred_element_type=_f32) + b1f_r[...])
            gb = (lax.dot_general(x_r[tb], w1bi_r[...], dn, preferred_element_type=_f32)
                  + jnp.dot(hb, w1bh_r[...], preferred_element_type=_f32) + b1b_r[...])
            hf, cf = cell(gf, cf)
            hb, cb = cell(gb, cb)
            hs_r[t, :, 0:H] = hf
            hs_r[tb, :, H:2 * H] = hb
            return (hf, cf, hb, cb)

        hf1, _, hb1, _ = lax.fori_loop(0, T, step1, (z, z, z, z))
        out_r[0] = hf1
        out_r[1] = hb1

        def step2(t, c4):
            hf, cf, hb, cb = c4
            tb = 47 - t
            gf = (jnp.dot(hs_r[t], w2fi_r[...], preferred_element_type=_f32)
                  + jnp.dot(hf, w2fh_r[...], preferred_element_type=_f32) + b2f_r[...])
            gb = (jnp.dot(hs_r[tb], w2bi_r[...], preferred_element_type=_f32)
                  + jnp.dot(hb, w2bh_r[...], preferred_element_type=_f32) + b2b_r[...])
            hf, cf = cell(gf, cf)
            hb, cb = cell(gb, cb)
            return (hf, cf, hb, cb)

        hf2, _, hb2, _ = lax.fori_loop(0, T, step2, (z, z, z, z))
        out_r[2] = hf2
        out_r[3] = hb2

    return pl.pallas_call(
        body,
        out_shape=jax.ShapeDtypeStruct((4, NP, H), _f32),
        scratch_shapes=[pltpu.VMEM((T, NP, 2 * H), _f32)],
    )(x2p, w1fi, w1fh, b1f, w1bi, w1bh, b1b, w2fi, w2fh, b2f, w2bi, w2bh, b2b)


def _head_body(xcat_ref, g_ref, b_ref,
               aW1_ref, ab1_ref, aW2_ref, ab2_ref, aW3_ref, ab3_ref,
               cW1_ref, cb1_ref, cW2_ref, cb2_ref, cW3_ref, cb3_ref,
               act_ref, con_ref):
    x = xcat_ref[...]
    mu = jnp.mean(x, axis=0, keepdims=True)
    var = jnp.mean((x - mu) * (x - mu), axis=0, keepdims=True)
    x3 = jnp.tanh((x - mu) / jnp.sqrt(var + 1e-5) * g_ref[...] + b_ref[...])

    def head(w1, c1, w2, c2, w3, c3):
        h = jnp.tanh(jnp.dot(x3, w1[...], preferred_element_type=_f32) + c1[...])
        h = jnp.tanh(jnp.dot(h, w2[...], preferred_element_type=_f32) + c2[...])
        return jnp.dot(h, w3[...], preferred_element_type=_f32) + c3[...]

    act_ref[...] = head(aW1_ref, ab1_ref, aW2_ref, ab2_ref, aW3_ref, ab3_ref)
    con_ref[...] = head(cW1_ref, cb1_ref, cW2_ref, cb2_ref, cW3_ref, cb3_ref)


def _heads(xcat, p):
    args = [xcat,
            p["bn3_g"].reshape(1, LIN), p["bn3_b"].reshape(1, LIN),
            p["aW1"], p["ab1"].reshape(1, LIN), p["aW2"], p["ab2"].reshape(1, LIN),
            p["aW3"], p["ab3"].reshape(1, 1),
            p["cW1"], p["cb1"].reshape(1, LIN), p["cW2"], p["cb2"].reshape(1, LIN),
            p["cW3"], p["cb3"].reshape(1, 1)]
    return pl.pallas_call(
        _head_body,
        out_shape=[jax.ShapeDtypeStruct((NNODES, 1), _f32),
                   jax.ShapeDtypeStruct((NNODES, 1), _f32)],
    )(*args)


def kernel(feat, node_id, edge_index, params):
    p = params
    src = edge_index[0].astype(jnp.int32)
    dst = edge_index[1].astype(jnp.int32)

    padn = EP - E
    pidx = jnp.arange(padn, dtype=jnp.int32)
    sfg = jnp.concatenate([src, pidx % 1024])
    ddm = jnp.concatenate([dst, N + (pidx % 64)])
    sdm = jnp.concatenate([src, N + (pidx % 64)])

    gidx1 = (2 * sfg)[None, :] + jnp.array([[0], [1]], jnp.int32)
    gidx1 = gidx1.reshape(2, 16, NG1, KCH, 128)
    didx1 = jnp.broadcast_to(ddm, (2, EP)).reshape(2, 16, NG1, KCH, 128)
    gidx2 = sfg.reshape(2, 16, NG2, KCH, 128)
    didx2 = ddm.reshape(2, 16, NG2, KCH, 128)
    didxd = jnp.stack([sdm, ddm]).reshape(2, 16, NG1, KCH, 128)

    nidp = jnp.concatenate(
        [node_id.astype(jnp.int32),
         jnp.arange(NODEP - N, dtype=jnp.int32) % NNODES]).reshape(32, NPB, 128)

    zeros24 = jnp.zeros((NR, 24), _f32)
    zeros8 = jnp.zeros((NR, 8), _f32)
    ones8 = jnp.ones((128, 8), _f32)

    emb16 = jnp.pad(p["emb"], ((0, 0), (0, 16 - EMB)))
    degs, embg_full = _sc_deg_emb(didxd, ones8, zeros8, nidp, emb16)
    embg = embg_full[:N, :EMB]
    dego = degs[0, :N]
    degi = degs[1, :N]

    hn1 = _tc_hn1(feat, embg, dego, p["W1"])

    eagg1 = _sc_agg(hn1.reshape(2 * N, 24), gidx1, didx1, zeros24, NG1)
    e10 = eagg1[0, :N]
    e11 = eagg1[1, :N]

    b1 = p["b1"].reshape(1, OUT1)
    s1, q1 = _tc_stats1(e10, e11, hn1, degi, b1)
    hn2 = _tc_apply1(e10, e11, hn1, degi, dego, b1,
                     p["bn1_g"].reshape(1, OUT1), p["bn1_b"].reshape(1, OUT1),
                     p["W2"], s1, q1)

    eagg2 = _sc_agg(hn2, gidx2, didx2, zeros24, NG2)
    e20 = eagg2[0, :N]
    e21 = eagg2[1, :N]

    b2 = p["b2"].reshape(1, OUT2)
    s2, q2 = _tc_stats2(e20, e21, hn2, degi, b2)
    x2 = _tc_apply2(e20, e21, hn2, degi, b2,
                    p["bn2_g"].reshape(1, OUT2), p["bn2_b"].reshape(1, OUT2),
                    s2, q2)

    x2p = jnp.pad(x2.reshape(T, NNODES, OUT2).transpose(0, 2, 1),
                  ((0, 0), (0, 0), (0, NP - NNODES)))

    lw = p["lstm"]
    ws = []
    for li in range(2):
        for d in range(2):
            pd = lw[li][d]
            ws += [pd["Wih"].T, pd["Whh"].T,
                   (pd["bih"] + pd["bhh"]).reshape(1, 4 * H)]
    finals = _tc_lstm(x2p, ws)
    h_n = finals[:, :NNODES, :].reshape(NNODES, 4 * H)

    x_last = jnp.concatenate([feat[-NNODES:], embg[-NNODES:]], axis=1)
    xcat = jnp.concatenate([x_last, h_n], axis=1)
    active, consume = _heads(xcat, p)
    return (active, consume)
